# baseline (speedup 1.0000x reference)
import jax
import jax.numpy as jnp
from jax.experimental import pallas as pl
from jax.experimental.pallas import tpu as pltpu

_B, _C_IN, _L = 16, 64, 4096
_CODE_DIM, _K = 32, 512
_LT = 4096


def _vq_body(x_ref, we_ref, be_ref, cb_ref, cbt_ref, cbsq_ref, wd_ref, bd_ref,
             recon_ref, zq_ref, z_ref):
    xt = x_ref[0]
    z = jnp.dot(we_ref[...], xt, preferred_element_type=jnp.float32) + be_ref[...]
    z_ref[0] = z
    row_sq = jnp.sum(z * z, axis=0, keepdims=True)
    cross2 = jnp.dot(cb_ref[...], z + z, preferred_element_type=jnp.float32)
    d2 = (row_sq - cross2) + cbsq_ref[...]
    m = jnp.min(d2, axis=0, keepdims=True)
    kiota = jax.lax.broadcasted_iota(jnp.int32, d2.shape, 0)
    cand = jnp.where(d2 == m, kiota, _K)
    idx = jnp.min(cand, axis=0, keepdims=True)
    onehot = (cand == idx).astype(jnp.float32)
    zq = jnp.dot(cbt_ref[...], onehot, preferred_element_type=jnp.float32)
    zq_ref[0] = zq
    recon_ref[0] = (
        jnp.dot(wd_ref[...], zq, preferred_element_type=jnp.float32) + bd_ref[...]
    )


def kernel(x, W_enc, b_enc, codebook, W_dec, b_dec):
    cb_sq = jnp.sum(codebook * codebook, axis=1)[:, None]
    grid = (_B, _L // _LT)
    out = pl.pallas_call(
        _vq_body,
        grid=grid,
        in_specs=[
            pl.BlockSpec((1, _C_IN, _LT), lambda b, l: (b, 0, l)),
            pl.BlockSpec((_CODE_DIM, _C_IN), lambda b, l: (0, 0)),
            pl.BlockSpec((_CODE_DIM, 1), lambda b, l: (0, 0)),
            pl.BlockSpec((_K, _CODE_DIM), lambda b, l: (0, 0)),
            pl.BlockSpec((_CODE_DIM, _K), lambda b, l: (0, 0)),
            pl.BlockSpec((_K, 1), lambda b, l: (0, 0)),
            pl.BlockSpec((_C_IN, _CODE_DIM), lambda b, l: (0, 0)),
            pl.BlockSpec((_C_IN, 1), lambda b, l: (0, 0)),
        ],
        out_specs=[
            pl.BlockSpec((1, _C_IN, _LT), lambda b, l: (b, 0, l)),
            pl.BlockSpec((1, _CODE_DIM, _LT), lambda b, l: (b, 0, l)),
            pl.BlockSpec((1, _CODE_DIM, _LT), lambda b, l: (b, 0, l)),
        ],
        out_shape=[
            jax.ShapeDtypeStruct((_B, _C_IN, _L), jnp.float32),
            jax.ShapeDtypeStruct((_B, _CODE_DIM, _L), jnp.float32),
            jax.ShapeDtypeStruct((_B, _CODE_DIM, _L), jnp.float32),
        ],
        compiler_params=pltpu.CompilerParams(
            dimension_semantics=("parallel", "parallel"),
        ),
    )(x, W_enc, b_enc[:, None], codebook, codebook.T, cb_sq, W_dec, b_dec[:, None])
    recon, z_q, z = out
    return (recon, z_q, z)


# speedup vs baseline: 1.0631x; 1.0631x over previous
import jax
import jax.numpy as jnp
from jax.experimental import pallas as pl
from jax.experimental.pallas import tpu as pltpu

_B, _C_IN, _L = 16, 64, 4096
_CODE_DIM, _K = 32, 512
_LT = 4096


def _vq_body(x_ref, we_ref, be_ref, cb_ref, cbt_ref, cbsq_ref, wd_ref, bd_ref,
             recon_ref, zq_ref, z_ref):
    xt = x_ref[0]
    z = jnp.dot(we_ref[...], xt, preferred_element_type=jnp.float32) + be_ref[...]
    z_ref[0] = z
    row_sq = jnp.sum(z * z, axis=0, keepdims=True)
    cross2 = jnp.dot(cb_ref[...], z + z, preferred_element_type=jnp.float32)
    d2 = (row_sq - cross2) + cbsq_ref[...]
    m = jnp.min(d2, axis=0, keepdims=True)
    kiota = jax.lax.broadcasted_iota(jnp.int32, d2.shape, 0)
    cand = jnp.where(d2 == m, kiota, _K)
    idx = jnp.min(cand, axis=0, keepdims=True)
    onehot = (kiota == idx).astype(jnp.float32)
    zq = jnp.dot(cbt_ref[...], onehot, preferred_element_type=jnp.float32)
    zq_ref[0] = zq
    recon_ref[0] = (
        jnp.dot(wd_ref[...], zq, preferred_element_type=jnp.float32) + bd_ref[...]
    )


def kernel(x, W_enc, b_enc, codebook, W_dec, b_dec):
    cb_sq = jnp.sum(codebook * codebook, axis=1)[:, None]
    grid = (_B, _L // _LT)
    out = pl.pallas_call(
        _vq_body,
        grid=grid,
        in_specs=[
            pl.BlockSpec((1, _C_IN, _LT), lambda b, l: (b, 0, l)),
            pl.BlockSpec((_CODE_DIM, _C_IN), lambda b, l: (0, 0)),
            pl.BlockSpec((_CODE_DIM, 1), lambda b, l: (0, 0)),
            pl.BlockSpec((_K, _CODE_DIM), lambda b, l: (0, 0)),
            pl.BlockSpec((_CODE_DIM, _K), lambda b, l: (0, 0)),
            pl.BlockSpec((_K, 1), lambda b, l: (0, 0)),
            pl.BlockSpec((_C_IN, _CODE_DIM), lambda b, l: (0, 0)),
            pl.BlockSpec((_C_IN, 1), lambda b, l: (0, 0)),
        ],
        out_specs=[
            pl.BlockSpec((1, _C_IN, _LT), lambda b, l: (b, 0, l)),
            pl.BlockSpec((1, _CODE_DIM, _LT), lambda b, l: (b, 0, l)),
            pl.BlockSpec((1, _CODE_DIM, _LT), lambda b, l: (b, 0, l)),
        ],
        out_shape=[
            jax.ShapeDtypeStruct((_B, _C_IN, _L), jnp.float32),
            jax.ShapeDtypeStruct((_B, _CODE_DIM, _L), jnp.float32),
            jax.ShapeDtypeStruct((_B, _CODE_DIM, _L), jnp.float32),
        ],
        compiler_params=pltpu.CompilerParams(
            dimension_semantics=("parallel", "parallel"),
        ),
    )(x, W_enc, b_enc[:, None], codebook, codebook.T, cb_sq, W_dec, b_dec[:, None])
    recon, z_q, z = out
    return (recon, z_q, z)
